# Initial kernel scaffold; baseline (speedup 1.0000x reference)
import jax
import jax.numpy as jnp
from jax.experimental import pallas as pl


def kernel(indices, offsets, W, w_ih0, w_hh0, b_ih0, b_hh0, w_ih, w_hh, b_ih, b_hh):
    raise NotImplementedError("write your pallas kernel here")


# speedup vs baseline: 47.9804x; 47.9804x over previous
import functools

import jax
import jax.numpy as jnp
from jax import lax
from jax.experimental import pallas as pl
from jax.experimental.pallas import tpu as pltpu
from jax.experimental.pallas import tpu_sc as plsc

_VOCAB = 1000000
_DIM = 100
_DP = 112
_NW = 4096
_TOT = 32768
_NL = 8
_NTILES = 32
_WPT = _NW // _NTILES
_CE = 64
_VR = (_VOCAB * _DIM) // 128
_OFFPAD = 256

_mesh = plsc.VectorSubcoreMesh(core_axis_name="c", subcore_axis_name="s")


@functools.partial(
    pl.kernel,
    mesh=_mesh,
    out_type=jax.ShapeDtypeStruct((_NW // 8, 128), jnp.float32),
    scratch_types=[
        pltpu.VMEM((_OFFPAD,), jnp.int32),
        pltpu.VMEM((256,), jnp.int32),
        pltpu.VMEM((128,), jnp.int32),
        pltpu.VMEM((256,), jnp.int32),
        pltpu.VMEM((128, 128), jnp.float32),
        pltpu.VMEM((256,), jnp.float32),
        pltpu.VMEM((_WPT, _DP), jnp.float32),
        pltpu.VMEM((3, _DP), jnp.float32),
        pltpu.VMEM((16,), jnp.float32),
        pltpu.VMEM((_WPT // 8, 128), jnp.float32),
        pltpu.VMEM((32,), jnp.float32),
        pltpu.SemaphoreType.DMA,
    ],
)
def _embed_bag(idx_hbm, offp_hbm, w_hbm, wih_hbm, b_hbm, gi_hbm,
               off_l, dup_v, idx2_v, offs_v, rows_v, win_s, acc, wih_v, b_v,
               gi_v, red, sem):
    wid = lax.axis_index("s") * 2 + lax.axis_index("c")
    w0 = wid * _WPT

    pltpu.sync_copy(offp_hbm.at[pl.ds(w0, _OFFPAD)], off_l)
    pltpu.sync_copy(wih_hbm, wih_v)
    pltpu.sync_copy(b_hbm, b_v)

    e_start = off_l[pl.ds(0, 16)][0]
    e_end = off_l[pl.ds(_WPT - 8, 16)][8]

    zeros16 = jnp.zeros((16,), jnp.float32)

    def zero_body(w, carry):
        for k in range(_DP // 16):
            acc[w, pl.ds(k * 16, 16)] = zeros16
        return carry

    lax.fori_loop(0, _WPT, zero_body, 0)

    li = lax.iota(jnp.int32, 16)
    lane_hi4 = li >= 12

    a0 = e_start & -8
    nchunks = jnp.right_shift(e_end - a0 + _CE - 1, 6)
    parity = li & 1

    def chunk_body(c, carry):
        base = jnp.minimum(a0 + c * _CE, _TOT - _CE)
        cstart = a0 + c * _CE
        dstart = pl.multiple_of(2 * base, 16)
        pltpu.sync_copy(idx_hbm.at[pl.ds(dstart, 2 * _CE)],
                        dup_v.at[pl.ds(0, 2 * _CE)])
        for g in range(8):
            iv = dup_v[pl.ds(16 * g, 16)]
            fr = iv * _DIM
            vr = jnp.minimum(jnp.right_shift(fr, 7) + parity, _VR - 1)
            idx2_v[pl.ds(16 * g, 16)] = vr
            offs_v[pl.ds(16 * g, 16)] = fr & 127
        pltpu.async_copy(w_hbm.at[idx2_v], rows_v, sem).wait()

        def word_body(w, carry2):
            opair = off_l[pl.ds(w, 16)]
            lo_e = jnp.maximum(opair[0], cstart)
            hi_e = jnp.minimum(opair[1], base + _CE)

            @pl.when(hi_e > lo_e)
            def _():
                def elem_body(p, regs):
                    j = p - base
                    off = offs_v[pl.ds(2 * j, 16)][0]
                    for t in range(8):
                        win_s[pl.ds(16 * t, 16)] = rows_v[2 * j,
                                                          pl.ds(16 * t, 16)]
                    for t in range(8):
                        win_s[pl.ds(128 + 16 * t, 16)] = rows_v[
                            2 * j + 1, pl.ds(16 * t, 16)]
                    new = [regs[k] + win_s[pl.ds(off + 16 * k, 16)]
                           for k in range(6)]
                    tail = jnp.where(lane_hi4,
                                     win_s[pl.ds(off + 84, 16)], 0.0)
                    new.append(regs[6] + tail)
                    return tuple(new)

                regs0 = tuple(acc[w, pl.ds(k * 16, 16)] for k in range(7))
                regs = lax.fori_loop(lo_e, hi_e, elem_body, regs0)
                for k in range(7):
                    acc[w, pl.ds(k * 16, 16)] = regs[k]

            return carry2

        return lax.fori_loop(0, _WPT, word_body, carry)

    lax.fori_loop(0, nchunks, chunk_body, 0)

    b_reg = b_v[...]
    w_regs = [[wih_v[g, pl.ds(k * 16, 16)] for k in range(_DP // 16)]
              for g in range(3)]
    lane0 = li == 0
    lane1 = li == 1
    lane2 = li == 2
    red[pl.ds(16, 16)] = zeros16

    def _lanesum(vec):
        cur = vec
        for sh in (8, 4, 2, 1):
            red[pl.ds(0, 16)] = cur
            cur = cur + red[pl.ds(sh, 16)]
        return cur[0]

    def proj_body(w, carry):
        opair = off_l[pl.ds(w, 16)]
        cntv = jnp.full((16,), opair[1] - opair[0], jnp.int32)
        invv = 1.0 / jnp.maximum(cntv.astype(jnp.float32), 1.0)
        a_sl = [acc[w, pl.ds(k * 16, 16)] for k in range(_DP // 16)]
        dots = []
        for g in range(3):
            d = w_regs[g][0] * a_sl[0]
            for k in range(1, _DP // 16):
                d = d + w_regs[g][k] * a_sl[k]
            dots.append(_lanesum(d))
        row = jnp.where(lane0, dots[0],
                        jnp.where(lane1, dots[1],
                                  jnp.where(lane2, dots[2], 0.0)))
        gi_v[jnp.right_shift(w, 3), pl.ds((w & 7) * 16, 16)] = (
            row * invv + b_reg)
        return carry

    lax.fori_loop(0, _WPT, proj_body, 0)

    grow0 = pl.multiple_of(wid * (_WPT // 8), 8)
    pltpu.sync_copy(gi_v, gi_hbm.at[pl.ds(grow0, _WPT // 8), :])


@functools.partial(
    pl.kernel,
    mesh=_mesh,
    out_type=[jax.ShapeDtypeStruct((_NW,), jnp.float32),
              jax.ShapeDtypeStruct((16,), jnp.float32)],
    scratch_types=[
        pltpu.VMEM((_NW // 8, 128), jnp.float32),
        pltpu.VMEM((12, 16), jnp.float32),
        pltpu.VMEM((_NW + 128,), jnp.float32),
        pltpu.VMEM((16,), jnp.float32),
        pltpu.VMEM((32,), jnp.float32),
        pltpu.SemaphoreType.DMA,
    ],
)
def _gru_stack(gi_hbm, pk_hbm, y_hbm, hn_hbm, gi_v, pk_v, y_v, hn_v, h_s, sem):
    wid = lax.axis_index("s") * 2 + lax.axis_index("c")

    @pl.when(wid == 0)
    def _():
        pltpu.sync_copy(gi_hbm, gi_v)
        pltpu.sync_copy(pk_hbm, pk_v)
        li = lax.iota(jnp.int32, 16)
        h_s[pl.ds(0, 16)] = jnp.zeros((16,), jnp.float32)
        h_s[pl.ds(16, 16)] = jnp.zeros((16,), jnp.float32)
        u_r = pk_v[0, :]; u_z = pk_v[1, :]; u_n = pk_v[2, :]
        c_r = pk_v[3, :]; c_z = pk_v[4, :]; c_n = pk_v[5, :]
        a_r = pk_v[6, :]; a_z = pk_v[7, :]; a_n = pk_v[8, :]
        d_r = pk_v[9, :]; d_z = pk_v[10, :]; d_n = pk_v[11, :]
        lane0 = li == 0
        lane_layer = li < _NL
        one = jnp.float32(1.0)

        def step(s, h):
            x = h_s[pl.ds(7, 16)]
            s_a = jnp.minimum(s, _NW - 1)
            grow = gi_v[jnp.right_shift(s_a, 3), pl.ds((s_a & 7) * 16, 16)]
            gr = jnp.where(lane0, grow[0], u_r * x + c_r)
            gz = jnp.where(lane0, grow[1], u_z * x + c_z)
            gn = jnp.where(lane0, grow[2], u_n * x + c_n)
            r = one / (one + jnp.exp(-(gr + a_r * h + d_r)))
            z = one / (one + jnp.exp(-(gz + a_z * h + d_z)))
            u = gn + r * (a_n * h + d_n)
            n = one - 2.0 / (jnp.exp(2.0 * u) + one)
            hn = (one - z) * n + z * h
            t = s - li
            valid = (t >= 0) & (t < _NW) & lane_layer
            hn = jnp.where(valid, hn, h)
            h_s[pl.ds(8, 16)] = hn
            yrow = h_s[pl.ds(15, 16)]
            ti = jnp.maximum(s - (_NL - 1), 0)
            y_v[pl.ds(ti, 16)] = yrow
            return hn

        h_fin = lax.fori_loop(0, _NW + _NL - 1, step,
                              jnp.zeros((16,), jnp.float32))
        hn_v[...] = h_fin
        pltpu.sync_copy(y_v.at[pl.ds(0, _NW)], y_hbm)
        pltpu.sync_copy(hn_v, hn_hbm)


def kernel(indices, offsets, W, w_ih0, w_hh0, b_ih0, b_hh0, w_ih, w_hh, b_ih, b_hh):
    indices = indices.astype(jnp.int32)
    offsets = offsets.astype(jnp.int32)
    idx_dup = jnp.repeat(indices, 2)
    w_view = W.reshape(_VR, 128)
    offp = jnp.concatenate(
        [offsets, jnp.full((_OFFPAD - _WPT,), _TOT, jnp.int32)])
    wih_pad = jnp.zeros((3, _DP), jnp.float32)
    wih_pad = wih_pad.at[:, :96].set(w_ih0[:, :96])
    wih_pad = wih_pad.at[:, 108:112].set(w_ih0[:, 96:100])
    b_pad = jnp.zeros((16,), jnp.float32).at[:3].set(b_ih0)

    gi0 = _embed_bag(idx_dup, offp, w_view, wih_pad, b_pad)

    pk = jnp.zeros((12, 16), jnp.float32)
    pk = pk.at[0:3, 1:_NL].set(w_ih[:, :, 0].T)
    pk = pk.at[3:6, 1:_NL].set(b_ih.T)
    pk = pk.at[6:9, 0].set(w_hh0[:, 0])
    pk = pk.at[6:9, 1:_NL].set(w_hh[:, :, 0].T)
    pk = pk.at[9:12, 0].set(b_hh0)
    pk = pk.at[9:12, 1:_NL].set(b_hh.T)

    y, h_fin = _gru_stack(gi0, pk)
    return y.reshape(_NW, 1, 1), h_fin[:_NL].reshape(_NL, 1, 1)


# speedup vs baseline: 176.3606x; 3.6757x over previous
import functools

import jax
import jax.numpy as jnp
from jax import lax
from jax.experimental import pallas as pl
from jax.experimental.pallas import tpu as pltpu
from jax.experimental.pallas import tpu_sc as plsc

_VOCAB = 1000000
_DIM = 100
_DP = 112
_NW = 4096
_TOT = 32768
_NL = 8
_NTILES = 32
_WPT = _NW // _NTILES
_C = 128
_OFFPAD = 256

_mesh = plsc.VectorSubcoreMesh(core_axis_name="c", subcore_axis_name="s")


@functools.partial(
    pl.kernel,
    mesh=_mesh,
    out_type=jax.ShapeDtypeStruct((_NW // 8, 128), jnp.float32),
    scratch_types=[
        pltpu.VMEM((_OFFPAD,), jnp.int32),
        pltpu.VMEM((_C,), jnp.int32),
        pltpu.VMEM((_C, _DIM), jnp.float32),
        pltpu.VMEM((_WPT, _DP), jnp.float32),
        pltpu.VMEM((3, _DP), jnp.float32),
        pltpu.VMEM((16,), jnp.float32),
        pltpu.VMEM((_WPT // 8, 128), jnp.float32),
        pltpu.VMEM((32,), jnp.float32),
        pltpu.SemaphoreType.DMA,
    ],
)
def _embed_bag(idx_hbm, offp_hbm, w_hbm, wih_hbm, b_hbm, gi_hbm,
               off_l, idx_v, rows_v, acc, wih_v, b_v, gi_v, red, sem):
    wid = lax.axis_index("s") * 2 + lax.axis_index("c")
    w0 = wid * _WPT

    pltpu.sync_copy(offp_hbm.at[pl.ds(w0, _OFFPAD)], off_l)
    pltpu.sync_copy(wih_hbm, wih_v)
    pltpu.sync_copy(b_hbm, b_v)

    e_start = off_l[pl.ds(0, 16)][0]
    e_end = off_l[pl.ds(_WPT - 8, 16)][8]

    zeros16 = jnp.zeros((16,), jnp.float32)

    def zero_body(w, carry):
        for k in range(_DP // 16):
            acc[w, pl.ds(k * 16, 16)] = zeros16
        return carry

    lax.fori_loop(0, _WPT, zero_body, 0)

    li = lax.iota(jnp.int32, 16)
    lane_hi4 = li >= 12

    a0 = e_start & -8
    nchunks = jnp.right_shift(e_end - a0 + _C - 1, 7)

    def chunk_body(c, carry):
        base = jnp.minimum(a0 + c * _C, _TOT - _C)
        cstart = a0 + c * _C
        bstart = pl.multiple_of(base, 8)
        pltpu.sync_copy(idx_hbm.at[pl.ds(bstart, _C)], idx_v)
        for g in range(_C // 16):
            iv = idx_v[pl.ds(16 * g, 16)]
            copies = [
                pltpu.async_copy(w_hbm.at[iv[i], :],
                                 rows_v.at[16 * g + i, :], sem)
                for i in range(16)
            ]
            for cp in copies:
                cp.wait()

        def word_body(w, carry2):
            opair = off_l[pl.ds(w, 16)]
            lo_e = jnp.maximum(opair[0], cstart)
            hi_e = jnp.minimum(opair[1], base + _C)

            @pl.when(hi_e > lo_e)
            def _():
                def elem_body(p, regs):
                    j = p - base
                    new = [regs[k] + rows_v[j, pl.ds(k * 16, 16)]
                           for k in range(6)]
                    tail = jnp.where(lane_hi4, rows_v[j, pl.ds(84, 16)], 0.0)
                    new.append(regs[6] + tail)
                    return tuple(new)

                regs0 = tuple(acc[w, pl.ds(k * 16, 16)] for k in range(7))
                regs = lax.fori_loop(lo_e, hi_e, elem_body, regs0)
                for k in range(7):
                    acc[w, pl.ds(k * 16, 16)] = regs[k]

            return carry2

        return lax.fori_loop(0, _WPT, word_body, carry)

    lax.fori_loop(0, nchunks, chunk_body, 0)

    b_reg = b_v[...]
    w_regs = [[wih_v[g, pl.ds(k * 16, 16)] for k in range(_DP // 16)]
              for g in range(3)]
    lane0 = li == 0
    lane1 = li == 1
    lane2 = li == 2
    red[pl.ds(16, 16)] = zeros16

    def _lanesum(vec):
        cur = vec
        for sh in (8, 4, 2, 1):
            red[pl.ds(0, 16)] = cur
            cur = cur + red[pl.ds(sh, 16)]
        return cur[0]

    def proj_body(w, carry):
        opair = off_l[pl.ds(w, 16)]
        cntv = jnp.full((16,), opair[1] - opair[0], jnp.int32)
        invv = 1.0 / jnp.maximum(cntv.astype(jnp.float32), 1.0)
        a_sl = [acc[w, pl.ds(k * 16, 16)] for k in range(_DP // 16)]
        dots = []
        for g in range(3):
            d = w_regs[g][0] * a_sl[0]
            for k in range(1, _DP // 16):
                d = d + w_regs[g][k] * a_sl[k]
            dots.append(_lanesum(d))
        row = jnp.where(lane0, dots[0],
                        jnp.where(lane1, dots[1],
                                  jnp.where(lane2, dots[2], 0.0)))
        gi_v[jnp.right_shift(w, 3), pl.ds((w & 7) * 16, 16)] = (
            row * invv + b_reg)
        return carry

    lax.fori_loop(0, _WPT, proj_body, 0)

    grow0 = pl.multiple_of(wid * (_WPT // 8), 8)
    pltpu.sync_copy(gi_v, gi_hbm.at[pl.ds(grow0, _WPT // 8), :])


@functools.partial(
    pl.kernel,
    mesh=_mesh,
    out_type=[jax.ShapeDtypeStruct((_NW,), jnp.float32),
              jax.ShapeDtypeStruct((16,), jnp.float32)],
    scratch_types=[
        pltpu.VMEM((_NW // 8, 128), jnp.float32),
        pltpu.VMEM((12, 16), jnp.float32),
        pltpu.VMEM((_NW + 128,), jnp.float32),
        pltpu.VMEM((16,), jnp.float32),
        pltpu.VMEM((32,), jnp.float32),
        pltpu.SemaphoreType.DMA,
    ],
)
def _gru_stack(gi_hbm, pk_hbm, y_hbm, hn_hbm, gi_v, pk_v, y_v, hn_v, h_s, sem):
    wid = lax.axis_index("s") * 2 + lax.axis_index("c")

    @pl.when(wid == 0)
    def _():
        pltpu.sync_copy(gi_hbm, gi_v)
        pltpu.sync_copy(pk_hbm, pk_v)
        li = lax.iota(jnp.int32, 16)
        h_s[pl.ds(0, 16)] = jnp.zeros((16,), jnp.float32)
        h_s[pl.ds(16, 16)] = jnp.zeros((16,), jnp.float32)
        u_r = pk_v[0, :]; u_z = pk_v[1, :]; u_n = pk_v[2, :]
        c_r = pk_v[3, :]; c_z = pk_v[4, :]; c_n = pk_v[5, :]
        a_r = pk_v[6, :]; a_z = pk_v[7, :]; a_n = pk_v[8, :]
        d_r = pk_v[9, :]; d_z = pk_v[10, :]; d_n = pk_v[11, :]
        lane0 = li == 0
        lane_layer = li < _NL
        one = jnp.float32(1.0)

        def step(s, h):
            x = h_s[pl.ds(7, 16)]
            s_a = jnp.minimum(s, _NW - 1)
            grow = gi_v[jnp.right_shift(s_a, 3), pl.ds((s_a & 7) * 16, 16)]
            gr = jnp.where(lane0, grow[0], u_r * x + c_r)
            gz = jnp.where(lane0, grow[1], u_z * x + c_z)
            gn = jnp.where(lane0, grow[2], u_n * x + c_n)
            r = one / (one + jnp.exp(-(gr + a_r * h + d_r)))
            z = one / (one + jnp.exp(-(gz + a_z * h + d_z)))
            u = gn + r * (a_n * h + d_n)
            n = one - 2.0 / (jnp.exp(2.0 * u) + one)
            hn = (one - z) * n + z * h
            t = s - li
            valid = (t >= 0) & (t < _NW) & lane_layer
            hn = jnp.where(valid, hn, h)
            h_s[pl.ds(8, 16)] = hn
            yrow = h_s[pl.ds(15, 16)]
            ti = jnp.maximum(s - (_NL - 1), 0)
            y_v[pl.ds(ti, 16)] = yrow
            return hn

        h_fin = lax.fori_loop(0, _NW + _NL - 1, step,
                              jnp.zeros((16,), jnp.float32))
        hn_v[...] = h_fin
        pltpu.sync_copy(y_v.at[pl.ds(0, _NW)], y_hbm)
        pltpu.sync_copy(hn_v, hn_hbm)


def kernel(indices, offsets, W, w_ih0, w_hh0, b_ih0, b_hh0, w_ih, w_hh, b_ih, b_hh):
    indices = indices.astype(jnp.int32)
    offsets = offsets.astype(jnp.int32)
    offp = jnp.concatenate(
        [offsets, jnp.full((_OFFPAD - _WPT,), _TOT, jnp.int32)])
    wih_pad = jnp.zeros((3, _DP), jnp.float32)
    wih_pad = wih_pad.at[:, :96].set(w_ih0[:, :96])
    wih_pad = wih_pad.at[:, 108:112].set(w_ih0[:, 96:100])
    b_pad = jnp.zeros((16,), jnp.float32).at[:3].set(b_ih0)

    gi0 = _embed_bag(indices, offp, W, wih_pad, b_pad)

    pk = jnp.zeros((12, 16), jnp.float32)
    pk = pk.at[0:3, 1:_NL].set(w_ih[:, :, 0].T)
    pk = pk.at[3:6, 1:_NL].set(b_ih.T)
    pk = pk.at[6:9, 0].set(w_hh0[:, 0])
    pk = pk.at[6:9, 1:_NL].set(w_hh[:, :, 0].T)
    pk = pk.at[9:12, 0].set(b_hh0)
    pk = pk.at[9:12, 1:_NL].set(b_hh.T)

    y, h_fin = _gru_stack(gi0, pk)
    return y.reshape(_NW, 1, 1), h_fin[:_NL].reshape(_NL, 1, 1)


# speedup vs baseline: 183.2764x; 1.0392x over previous
import functools

import jax
import jax.numpy as jnp
from jax import lax
from jax.experimental import pallas as pl
from jax.experimental.pallas import tpu as pltpu
from jax.experimental.pallas import tpu_sc as plsc

_VOCAB = 1000000
_DIM = 100
_DP = 112
_NW = 4096
_TOT = 32768
_NL = 8
_NTILES = 32
_WPT = _NW // _NTILES
_C = 128
_OFFPAD = 256

_mesh = plsc.VectorSubcoreMesh(core_axis_name="c", subcore_axis_name="s")


@functools.partial(
    pl.kernel,
    mesh=_mesh,
    out_type=jax.ShapeDtypeStruct((_NW // 8, 128), jnp.float32),
    scratch_types=[
        pltpu.VMEM((_OFFPAD,), jnp.int32),
        pltpu.VMEM((_C,), jnp.int32),
        pltpu.VMEM((_C, _DIM), jnp.float32),
        pltpu.VMEM((_WPT, _DP), jnp.float32),
        pltpu.VMEM((3, _DP), jnp.float32),
        pltpu.VMEM((16,), jnp.float32),
        pltpu.VMEM((_WPT // 8, 128), jnp.float32),
        pltpu.VMEM((32,), jnp.float32),
        pltpu.SemaphoreType.DMA,
    ],
)
def _embed_bag(idx_hbm, offp_hbm, w_hbm, wih_hbm, b_hbm, gi_hbm,
               off_l, idx_v, rows_v, acc, wih_v, b_v, gi_v, red, sem):
    wid = lax.axis_index("s") * 2 + lax.axis_index("c")
    w0 = wid * _WPT

    pltpu.sync_copy(offp_hbm.at[pl.ds(w0, _OFFPAD)], off_l)
    pltpu.sync_copy(wih_hbm, wih_v)
    pltpu.sync_copy(b_hbm, b_v)

    e_start = off_l[pl.ds(0, 16)][0]
    e_end = off_l[pl.ds(_WPT - 8, 16)][8]

    zeros16 = jnp.zeros((16,), jnp.float32)

    def zero_body(w, carry):
        for k in range(_DP // 16):
            acc[w, pl.ds(k * 16, 16)] = zeros16
        return carry

    lax.fori_loop(0, _WPT, zero_body, 0)

    li = lax.iota(jnp.int32, 16)
    lane_hi4 = li >= 12

    a0 = e_start & -8
    nchunks = jnp.right_shift(e_end - a0 + _C - 1, 7)

    def chunk_body(c, carry):
        base = jnp.minimum(a0 + c * _C, _TOT - _C)
        cstart = a0 + c * _C
        bstart = pl.multiple_of(base, 8)
        pltpu.sync_copy(idx_hbm.at[pl.ds(bstart, _C)], idx_v)
        prev = None
        for g in range(_C // 16):
            iv = idx_v[pl.ds(16 * g, 16)]
            cur = [
                pltpu.async_copy(w_hbm.at[iv[i], :],
                                 rows_v.at[16 * g + i, :], sem)
                for i in range(16)
            ]
            if prev is not None:
                for cp in prev:
                    cp.wait()
            prev = cur
        for cp in prev:
            cp.wait()

        def word_body(w, carry2):
            opair = off_l[pl.ds(w, 16)]
            lo_e = jnp.maximum(opair[0], cstart)
            hi_e = jnp.minimum(opair[1], base + _C)

            @pl.when(hi_e > lo_e)
            def _():
                def elem_body(p, regs):
                    j = p - base
                    new = [regs[k] + rows_v[j, pl.ds(k * 16, 16)]
                           for k in range(6)]
                    tail = jnp.where(lane_hi4, rows_v[j, pl.ds(84, 16)], 0.0)
                    new.append(regs[6] + tail)
                    return tuple(new)

                regs0 = tuple(acc[w, pl.ds(k * 16, 16)] for k in range(7))
                regs = lax.fori_loop(lo_e, hi_e, elem_body, regs0)
                for k in range(7):
                    acc[w, pl.ds(k * 16, 16)] = regs[k]

            return carry2

        return lax.fori_loop(0, _WPT, word_body, carry)

    lax.fori_loop(0, nchunks, chunk_body, 0)

    b_reg = b_v[...]
    w_regs = [[wih_v[g, pl.ds(k * 16, 16)] for k in range(_DP // 16)]
              for g in range(3)]
    lane0 = li == 0
    lane1 = li == 1
    lane2 = li == 2
    red[pl.ds(16, 16)] = zeros16

    def _lanesum(vec):
        cur = vec
        for sh in (8, 4, 2, 1):
            red[pl.ds(0, 16)] = cur
            cur = cur + red[pl.ds(sh, 16)]
        return cur[0]

    def proj_body(w, carry):
        opair = off_l[pl.ds(w, 16)]
        cntv = jnp.full((16,), opair[1] - opair[0], jnp.int32)
        invv = 1.0 / jnp.maximum(cntv.astype(jnp.float32), 1.0)
        a_sl = [acc[w, pl.ds(k * 16, 16)] for k in range(_DP // 16)]
        dots = []
        for g in range(3):
            d = w_regs[g][0] * a_sl[0]
            for k in range(1, _DP // 16):
                d = d + w_regs[g][k] * a_sl[k]
            dots.append(_lanesum(d))
        row = jnp.where(lane0, dots[0],
                        jnp.where(lane1, dots[1],
                                  jnp.where(lane2, dots[2], 0.0)))
        gi_v[jnp.right_shift(w, 3), pl.ds((w & 7) * 16, 16)] = (
            row * invv + b_reg)
        return carry

    lax.fori_loop(0, _WPT, proj_body, 0)

    grow0 = pl.multiple_of(wid * (_WPT // 8), 8)
    pltpu.sync_copy(gi_v, gi_hbm.at[pl.ds(grow0, _WPT // 8), :])


@functools.partial(
    pl.kernel,
    mesh=_mesh,
    out_type=[jax.ShapeDtypeStruct((_NW,), jnp.float32),
              jax.ShapeDtypeStruct((16,), jnp.float32)],
    scratch_types=[
        pltpu.VMEM((_NW // 8, 128), jnp.float32),
        pltpu.VMEM((12, 16), jnp.float32),
        pltpu.VMEM((_NW + 128,), jnp.float32),
        pltpu.VMEM((16,), jnp.float32),
        pltpu.VMEM((32,), jnp.float32),
        pltpu.SemaphoreType.DMA,
    ],
)
def _gru_stack(gi_hbm, pk_hbm, y_hbm, hn_hbm, gi_v, pk_v, y_v, hn_v, h_s, sem):
    wid = lax.axis_index("s") * 2 + lax.axis_index("c")

    @pl.when(wid == 0)
    def _():
        pltpu.sync_copy(gi_hbm, gi_v)
        pltpu.sync_copy(pk_hbm, pk_v)
        li = lax.iota(jnp.int32, 16)
        h_s[pl.ds(0, 16)] = jnp.zeros((16,), jnp.float32)
        h_s[pl.ds(16, 16)] = jnp.zeros((16,), jnp.float32)
        u_r = pk_v[0, :]; u_z = pk_v[1, :]; u_n = pk_v[2, :]
        c_r = pk_v[3, :]; c_z = pk_v[4, :]; c_n = pk_v[5, :]
        a_r = pk_v[6, :]; a_z = pk_v[7, :]; a_n = pk_v[8, :]
        d_r = pk_v[9, :]; d_z = pk_v[10, :]; d_n = pk_v[11, :]
        lane0 = li == 0
        lane_layer = li < _NL
        one = jnp.float32(1.0)

        def step(s, h):
            x = h_s[pl.ds(7, 16)]
            s_a = jnp.minimum(s, _NW - 1)
            grow = gi_v[jnp.right_shift(s_a, 3), pl.ds((s_a & 7) * 16, 16)]
            gr = jnp.where(lane0, grow[0], u_r * x + c_r)
            gz = jnp.where(lane0, grow[1], u_z * x + c_z)
            gn = jnp.where(lane0, grow[2], u_n * x + c_n)
            r = one / (one + jnp.exp(-(gr + a_r * h + d_r)))
            z = one / (one + jnp.exp(-(gz + a_z * h + d_z)))
            u = gn + r * (a_n * h + d_n)
            n = one - 2.0 / (jnp.exp(2.0 * u) + one)
            hn = (one - z) * n + z * h
            t = s - li
            valid = (t >= 0) & (t < _NW) & lane_layer
            hn = jnp.where(valid, hn, h)
            h_s[pl.ds(8, 16)] = hn
            yrow = h_s[pl.ds(15, 16)]
            ti = jnp.maximum(s - (_NL - 1), 0)
            y_v[pl.ds(ti, 16)] = yrow
            return hn

        h_fin = lax.fori_loop(0, _NW + _NL - 1, step,
                              jnp.zeros((16,), jnp.float32), unroll=4)
        hn_v[...] = h_fin
        pltpu.sync_copy(y_v.at[pl.ds(0, _NW)], y_hbm)
        pltpu.sync_copy(hn_v, hn_hbm)


def kernel(indices, offsets, W, w_ih0, w_hh0, b_ih0, b_hh0, w_ih, w_hh, b_ih, b_hh):
    indices = indices.astype(jnp.int32)
    offsets = offsets.astype(jnp.int32)
    offp = jnp.concatenate(
        [offsets, jnp.full((_OFFPAD - _WPT,), _TOT, jnp.int32)])
    wih_pad = jnp.zeros((3, _DP), jnp.float32)
    wih_pad = wih_pad.at[:, :96].set(w_ih0[:, :96])
    wih_pad = wih_pad.at[:, 108:112].set(w_ih0[:, 96:100])
    b_pad = jnp.zeros((16,), jnp.float32).at[:3].set(b_ih0)

    gi0 = _embed_bag(indices, offp, W, wih_pad, b_pad)

    pk = jnp.zeros((12, 16), jnp.float32)
    pk = pk.at[0:3, 1:_NL].set(w_ih[:, :, 0].T)
    pk = pk.at[3:6, 1:_NL].set(b_ih.T)
    pk = pk.at[6:9, 0].set(w_hh0[:, 0])
    pk = pk.at[6:9, 1:_NL].set(w_hh[:, :, 0].T)
    pk = pk.at[9:12, 0].set(b_hh0)
    pk = pk.at[9:12, 1:_NL].set(b_hh.T)

    y, h_fin = _gru_stack(gi0, pk)
    return y.reshape(_NW, 1, 1), h_fin[:_NL].reshape(_NL, 1, 1)
